# Initial kernel scaffold; baseline (speedup 1.0000x reference)
import functools

import jax
import jax.numpy as jnp
from jax import lax
from jax.experimental import pallas as pl
from jax.experimental.pallas import tpu as pltpu
from jax.experimental.pallas import tpu_sc as plsc

_NUM = 1000000
_RANK = 16
_OUT_DIM = 64
_B = 16384
_L = 50
_TOKENS = _B * _L

_info = plsc.get_sparse_core_info()
_NC = _info.num_cores
_NS = _info.num_subcores
_NW = _NC * _NS
_PER_W = _TOKENS // _NW
_CH = 128
_NCH = _PER_W // _CH


def _sc_gather(idx, table):
    mesh = plsc.VectorSubcoreMesh(core_axis_name="c", subcore_axis_name="s")

    @functools.partial(
        pl.kernel,
        mesh=mesh,
        out_type=jax.ShapeDtypeStruct((_TOKENS, _RANK), jnp.float32),
        scratch_types=[
            pltpu.VMEM((_NCH, _CH), jnp.int32),
            pltpu.VMEM((_CH, _RANK), jnp.float32),
            pltpu.SemaphoreType.DMA,
        ],
    )
    def gather_kernel(idx_hbm, table_hbm, out_hbm, idx_v, rows_v, sem):
        wid = lax.axis_index("s") * _NC + lax.axis_index("c")
        base = wid * _PER_W
        pltpu.sync_copy(idx_hbm.at[wid], idx_v)

        def body(j, carry):
            pltpu.async_copy(table_hbm.at[idx_v.at[j]], rows_v, sem).wait()
            pltpu.sync_copy(rows_v, out_hbm.at[pl.ds(base + j * _CH, _CH)])
            return carry

        lax.fori_loop(0, _NCH, body, 0, unroll=False)

    return gather_kernel(idx, table)


def _proj_block(e_ref, wt_ref, o_ref):
    o_ref[...] = jnp.dot(e_ref[...], wt_ref[...],
                         preferred_element_type=jnp.float32)


def _tc_project(rows, wt):
    blk = 8192
    grid = _TOKENS // blk
    return pl.pallas_call(
        _proj_block,
        grid=(grid,),
        in_specs=[
            pl.BlockSpec((blk, _RANK), lambda i: (i, 0)),
            pl.BlockSpec((_RANK, _OUT_DIM), lambda i: (0, 0)),
        ],
        out_specs=pl.BlockSpec((blk, _OUT_DIM), lambda i: (i, 0)),
        out_shape=jax.ShapeDtypeStruct((_TOKENS, _OUT_DIM), jnp.float32),
    )(rows, wt)


def kernel(src, emb_table, W):
    idx = src.reshape(_NW, _NCH, _CH)
    rows = _sc_gather(idx, emb_table)
    out = _tc_project(rows, W.T)
    return out.reshape(_B, _L, _OUT_DIM)


# speedup vs baseline: 9.9422x; 9.9422x over previous
import functools

import jax
import jax.numpy as jnp
from jax import lax
from jax.experimental import pallas as pl
from jax.experimental.pallas import tpu as pltpu
from jax.experimental.pallas import tpu_sc as plsc

_NUM = 1000000
_RANK = 16
_OUT_DIM = 64
_B = 16384
_L = 50
_TOKENS = _B * _L

_info = plsc.get_sparse_core_info()
_NC = _info.num_cores
_NS = _info.num_subcores
_NW = _NC * _NS
_PER_W = _TOKENS // _NW
_CH = 128
_NCH = _PER_W // _CH


def _sc_gather(idx, table):
    mesh = plsc.VectorSubcoreMesh(core_axis_name="c", subcore_axis_name="s")

    @functools.partial(
        pl.kernel,
        mesh=mesh,
        compiler_params=pltpu.CompilerParams(use_tc_tiling_on_sc=False),
        out_type=jax.ShapeDtypeStruct((_TOKENS, _RANK), jnp.float32),
        scratch_types=[
            pltpu.VMEM((_NCH, _CH), jnp.int32),
            pltpu.VMEM((_CH, _RANK), jnp.float32),
            pltpu.SemaphoreType.DMA,
        ],
    )
    def gather_kernel(idx_hbm, table_hbm, out_hbm, idx_v, rows_v, sem):
        wid = lax.axis_index("s") * _NC + lax.axis_index("c")
        base = wid * _PER_W
        pltpu.sync_copy(idx_hbm.at[wid], idx_v)

        def body(j, carry):
            pltpu.async_copy(table_hbm.at[idx_v.at[j]], rows_v, sem).wait()
            pltpu.sync_copy(rows_v, out_hbm.at[pl.ds(base + j * _CH, _CH)])
            return carry

        lax.fori_loop(0, _NCH, body, 0, unroll=False)

    return gather_kernel(idx, table)


def _proj_block(e_ref, wt_ref, o_ref):
    o_ref[...] = jnp.dot(e_ref[...], wt_ref[...],
                         preferred_element_type=jnp.float32)


def _tc_project(rows, wt):
    blk = 8192
    grid = _TOKENS // blk
    return pl.pallas_call(
        _proj_block,
        grid=(grid,),
        in_specs=[
            pl.BlockSpec((blk, _RANK), lambda i: (i, 0)),
            pl.BlockSpec((_RANK, _OUT_DIM), lambda i: (0, 0)),
        ],
        out_specs=pl.BlockSpec((blk, _OUT_DIM), lambda i: (i, 0)),
        out_shape=jax.ShapeDtypeStruct((_TOKENS, _OUT_DIM), jnp.float32),
    )(rows, wt)


def kernel(src, emb_table, W):
    idx = src.reshape(_NW, _NCH, _CH)
    rows = _sc_gather(idx, emb_table)
    out = _tc_project(rows, W.T)
    return out.reshape(_B, _L, _OUT_DIM)


# speedup vs baseline: 10.7335x; 1.0796x over previous
import functools

import jax
import jax.numpy as jnp
from jax import lax
from jax.experimental import pallas as pl
from jax.experimental.pallas import tpu as pltpu
from jax.experimental.pallas import tpu_sc as plsc

_NUM = 1000000
_RANK = 16
_OUT_DIM = 64
_B = 16384
_L = 50
_TOKENS = _B * _L

_info = plsc.get_sparse_core_info()
_NC = _info.num_cores
_NS = _info.num_subcores
_NW = _NC * _NS
_PER_W = _TOKENS // _NW
_CH = 128
_NCH = _PER_W // _CH
_K = 10
_RPB = _K * _CH
_ROUNDS = _NCH // _K


def _sc_gather(idx, table):
    mesh = plsc.VectorSubcoreMesh(core_axis_name="c", subcore_axis_name="s")

    @functools.partial(
        pl.kernel,
        mesh=mesh,
        compiler_params=pltpu.CompilerParams(use_tc_tiling_on_sc=False),
        out_type=jax.ShapeDtypeStruct((_TOKENS, _RANK), jnp.float32),
        scratch_types=[
            pltpu.VMEM((_NCH, _CH), jnp.int32),
            pltpu.VMEM((2, _RPB, _RANK), jnp.float32),
            pltpu.SemaphoreType.DMA,
            pltpu.SemaphoreType.DMA,
            pltpu.SemaphoreType.DMA,
        ],
    )
    def gather_kernel(idx_hbm, table_hbm, out_hbm, idx_v, rows_v, gsem,
                      wsem0, wsem1):
        wid = lax.axis_index("s") * _NC + lax.axis_index("c")
        base = wid * _PER_W
        pltpu.sync_copy(idx_hbm.at[wid], idx_v)
        wsems = (wsem0, wsem1)

        def outer(o, carry):
            for buf in range(2):
                r = o * 2 + buf

                @pl.when(o > 0)
                def _():
                    pltpu.make_async_copy(
                        rows_v.at[buf],
                        out_hbm.at[pl.ds(base, _RPB)],
                        wsems[buf],
                    ).wait()

                cps = [
                    pltpu.async_copy(
                        table_hbm.at[idx_v.at[r * _K + b]],
                        rows_v.at[buf, pl.ds(b * _CH, _CH)],
                        gsem,
                    )
                    for b in range(_K)
                ]
                for cp in cps:
                    cp.wait()
                pltpu.async_copy(
                    rows_v.at[buf],
                    out_hbm.at[pl.ds(base + r * _RPB, _RPB)],
                    wsems[buf],
                )
            return carry

        lax.fori_loop(0, _ROUNDS // 2, outer, 0, unroll=False)
        for buf in range(2):
            pltpu.make_async_copy(
                rows_v.at[buf],
                out_hbm.at[pl.ds(base, _RPB)],
                wsems[buf],
            ).wait()

    return gather_kernel(idx, table)


def _proj_block(e_ref, wt_ref, o_ref):
    o_ref[...] = jnp.dot(e_ref[...], wt_ref[...],
                         preferred_element_type=jnp.float32)


def _tc_project(rows, wt):
    blk = 8192
    grid = _TOKENS // blk
    return pl.pallas_call(
        _proj_block,
        grid=(grid,),
        in_specs=[
            pl.BlockSpec((blk, _RANK), lambda i: (i, 0)),
            pl.BlockSpec((_RANK, _OUT_DIM), lambda i: (0, 0)),
        ],
        out_specs=pl.BlockSpec((blk, _OUT_DIM), lambda i: (i, 0)),
        out_shape=jax.ShapeDtypeStruct((_TOKENS, _OUT_DIM), jnp.float32),
    )(rows, wt)


def kernel(src, emb_table, W):
    idx = src.reshape(_NW, _NCH, _CH)
    rows = _sc_gather(idx, emb_table)
    out = _tc_project(rows, W.T)
    return out.reshape(_B, _L, _OUT_DIM)


# speedup vs baseline: 15.4240x; 1.4370x over previous
import functools

import jax
import jax.numpy as jnp
from jax import lax
from jax.experimental import pallas as pl
from jax.experimental.pallas import tpu as pltpu
from jax.experimental.pallas import tpu_sc as plsc

_NUM = 1000000
_RANK = 16
_OUT_DIM = 64
_B = 16384
_L = 50
_TOKENS = _B * _L

_info = plsc.get_sparse_core_info()
_NC = _info.num_cores
_NS = _info.num_subcores
_NW = _NC * _NS
_PER_W = _TOKENS // _NW
_CH = 128
_NCH = _PER_W // _CH
_K = 10
_RPB = _K * _CH
_ROUNDS = _NCH // _K
_PACK = 8
_PROWS = _TOKENS // _PACK


def _sc_gather(idx, table):
    mesh = plsc.VectorSubcoreMesh(core_axis_name="c", subcore_axis_name="s")
    fpr = _RPB * _RANK

    @functools.partial(
        pl.kernel,
        mesh=mesh,
        compiler_params=pltpu.CompilerParams(use_tc_tiling_on_sc=False),
        out_type=jax.ShapeDtypeStruct((_TOKENS * _RANK,), jnp.float32),
        scratch_types=[
            pltpu.VMEM((_NCH, _CH), jnp.int32),
            pltpu.VMEM((2, _RPB, _RANK), jnp.float32),
            pltpu.VMEM((2, fpr), jnp.float32),
            pltpu.SemaphoreType.DMA,
            pltpu.SemaphoreType.DMA,
            pltpu.SemaphoreType.DMA,
            pltpu.SemaphoreType.DMA,
        ],
    )
    def gather_kernel(idx_hbm, table_hbm, out_hbm, idx_v, rows2d, rows1d,
                      gsem0, gsem1, wsem0, wsem1):
        wid = lax.axis_index("s") * _NC + lax.axis_index("c")
        fbase = wid * _PER_W * _RANK
        pltpu.sync_copy(idx_hbm.at[wid], idx_v)
        gsems = (gsem0, gsem1)
        wsems = (wsem0, wsem1)

        def fire(r, buf):
            for b in range(_K):
                pltpu.async_copy(
                    table_hbm.at[idx_v.at[r * _K + b]],
                    rows2d.at[buf, pl.ds(b * _CH, _CH)],
                    gsems[buf],
                )

        fire(0, 0)

        def outer(o, carry):
            for buf in range(2):
                r = o * 2 + buf
                nbuf = 1 - buf

                @pl.when(r + 1 < _ROUNDS)
                def _():
                    fire(r + 1, nbuf)

                pltpu.make_async_copy(
                    table_hbm.at[pl.ds(0, _RPB)],
                    rows2d.at[buf],
                    gsems[buf],
                ).wait()

                @pl.when(o > 0)
                def _():
                    pltpu.make_async_copy(
                        rows1d.at[buf],
                        out_hbm.at[pl.ds(fbase, fpr)],
                        wsems[buf],
                    ).wait()

                def repack(k, c):
                    rows1d[buf, pl.ds(k * _RANK, _RANK)] = rows2d[buf, k]
                    return c

                lax.fori_loop(0, _RPB, repack, 0, unroll=8)

                pltpu.async_copy(
                    rows1d.at[buf],
                    out_hbm.at[pl.ds(fbase + r * fpr, fpr)],
                    wsems[buf],
                )
            return carry

        lax.fori_loop(0, _ROUNDS // 2, outer, 0, unroll=False)
        for buf in range(2):
            pltpu.make_async_copy(
                rows1d.at[buf],
                out_hbm.at[pl.ds(fbase, fpr)],
                wsems[buf],
            ).wait()

    return gather_kernel(idx, table)


_BM = 128


def _proj_block(e_ref, wb_ref, o_ref):
    o_ref[...] = jnp.dot(e_ref[...], wb_ref[...],
                         preferred_element_type=jnp.float32)


def _tc_project(packed, wbig):
    blk = 2048
    grid = _PROWS // blk
    ndim = _PACK * _OUT_DIM
    return pl.pallas_call(
        _proj_block,
        grid=(grid,),
        in_specs=[
            pl.BlockSpec((blk, _PACK * _RANK), lambda i: (i, 0)),
            pl.BlockSpec((_PACK * _RANK, ndim), lambda i: (0, 0)),
        ],
        out_specs=pl.BlockSpec((blk, ndim), lambda i: (i, 0)),
        out_shape=jax.ShapeDtypeStruct((_PROWS, ndim), jnp.float32),
    )(packed, wbig)


def kernel(src, emb_table, W):
    idx = src.reshape(_NW, _NCH, _CH)
    flat = _sc_gather(idx, emb_table)
    packed = flat.reshape(_PROWS, _PACK * _RANK)
    wt = W.T
    eye = jnp.eye(_PACK, dtype=jnp.float32)
    wbig = jnp.einsum('pq,ro->prqo', eye, wt).reshape(
        _PACK * _RANK, _PACK * _OUT_DIM)
    return _tc_project(packed, wbig).reshape(_B, _L, _OUT_DIM)
